# baseline (speedup 1.0000x reference)
import jax
import jax.numpy as jnp
from jax import lax
from jax.experimental import pallas as pl
from jax.experimental.pallas import tpu as pltpu
from jax.experimental.pallas import tpu_sc as plsc

_B, _N, _D, _H = 512, 256, 128, 512
_K = 16
_L = 16
_NW = 32
_BB = 32


def _mlp_body(x_ref, w1_ref, b1_ref, w2_ref, b2_ref, s_ref, h_ref):
    logit = jnp.dot(h_ref[...], w2_ref[...], preferred_element_type=jnp.float32)
    logit = logit.reshape(s_ref.shape)
    s_ref[...] = jax.nn.sigmoid(logit + b2_ref[...]) - 0.5
    h = jnp.dot(x_ref[...], w1_ref[...], preferred_element_type=jnp.float32)
    h_ref[...] = jnp.maximum(h + b1_ref[...], 0.0)


def _mlp_scores(x_flat, W1, b1, W2, b2, nb):
    rows = _BB * _N
    nsteps = nb // _BB
    return pl.pallas_call(
        _mlp_body,
        grid=(nsteps + 1,),
        in_specs=[
            pl.BlockSpec((rows, _D), lambda i: (jnp.minimum(i, nsteps - 1), 0)),
            pl.BlockSpec((_D, _H), lambda i: (0, 0)),
            pl.BlockSpec((1, _H), lambda i: (0, 0)),
            pl.BlockSpec((_H, 1), lambda i: (0, 0)),
            pl.BlockSpec((1, 1), lambda i: (0, 0)),
        ],
        out_specs=pl.BlockSpec((_BB, _N), lambda i: (jnp.maximum(i - 1, 0), 0)),
        out_shape=jax.ShapeDtypeStruct((nb, _N), jnp.float32),
        scratch_shapes=[pltpu.VMEM((rows, _H), jnp.float32)],
    )(x_flat, W1, b1.reshape(1, _H), W2, b2.reshape(1, 1))


def _topk_body(s_hbm, out_hbm, s_v, out_v):
    wid = lax.axis_index("s") * 2 + lax.axis_index("c")
    rows = s_v.shape[0]
    base = wid * rows
    pltpu.sync_copy(s_hbm.at[pl.ds(base, rows)], s_v)
    zero = jnp.zeros((_L,), jnp.float32)
    for r in range(rows):
        ka = ia = None
        for c in range(_N // _L):
            vals = s_v[r, pl.ds(c * _L, _L)]
            kb = jnp.abs(vals)
            ib = lax.iota(jnp.int32, _L) + c * _L
            if c == 0:
                ka, ia = plsc.sort_key_val(kb, ib)
            else:
                kb, ib = plsc.sort_key_val(kb, ib, descending=True)
                take_a = ka >= kb
                km = jnp.where(take_a, ka, kb)
                im = jnp.where(take_a, ia, ib)
                ka, ia = plsc.sort_key_val(km, im)
        denom = jnp.sum(ka) + 1e-8
        row_splat = jnp.full((_L,), r, jnp.int32)
        sel = plsc.load_gather(s_v, [row_splat, ia])
        for c in range(_N // _L):
            out_v[r, pl.ds(c * _L, _L)] = zero
        plsc.store_scatter(out_v, [row_splat, ia], sel / denom)
    pltpu.sync_copy(out_v, out_hbm.at[pl.ds(base, rows)])


def _topk_sc(s2d):
    nb = s2d.shape[0]
    rows = nb // _NW
    f = pl.kernel(
        _topk_body,
        out_type=jax.ShapeDtypeStruct((nb, _N), jnp.float32),
        mesh=plsc.VectorSubcoreMesh(core_axis_name="c", subcore_axis_name="s"),
        compiler_params=pltpu.CompilerParams(needs_layout_passes=False),
        scratch_types=[
            pltpu.VMEM((rows, _N), jnp.float32),
            pltpu.VMEM((rows, _N), jnp.float32),
        ],
    )
    return f(s2d)


_NCHUNK = 1


def kernel(signal_features, W1, b1, W2, b2):
    nb = _B // _NCHUNK
    outs = []
    for c in range(_NCHUNK):
        x_flat = signal_features[c * nb:(c + 1) * nb].reshape(nb * _N, _D)
        s = _mlp_scores(x_flat, W1, b1, W2, b2, nb)
        outs.append(_topk_sc(s))
    return jnp.concatenate(outs, axis=0)


# speedup vs baseline: 1.0019x; 1.0019x over previous
import jax
import jax.numpy as jnp
from jax import lax
from jax.experimental import pallas as pl
from jax.experimental.pallas import tpu as pltpu
from jax.experimental.pallas import tpu_sc as plsc

_B, _N, _D, _H = 512, 256, 128, 512
_L = 16
_NW = 32
_BB = 32


def _mlp_body(x_ref, w1_ref, b1_ref, w2_ref, b2_ref, s_ref, h_ref):
    logit = jnp.dot(h_ref[...], w2_ref[...], preferred_element_type=jnp.float32)
    logit = logit.reshape(s_ref.shape)
    s_ref[...] = jax.nn.sigmoid(logit + b2_ref[...]) - 0.5
    h = jnp.dot(x_ref[...], w1_ref[...], preferred_element_type=jnp.float32)
    h_ref[...] = jnp.maximum(h + b1_ref[...], 0.0)


def _mlp_scores(x_flat, W1, b1, W2, b2, nb):
    rows = _BB * _N
    nsteps = nb // _BB
    return pl.pallas_call(
        _mlp_body,
        grid=(nsteps + 1,),
        in_specs=[
            pl.BlockSpec((rows, _D), lambda i: (jnp.minimum(i, nsteps - 1), 0)),
            pl.BlockSpec((_D, _H), lambda i: (0, 0)),
            pl.BlockSpec((1, _H), lambda i: (0, 0)),
            pl.BlockSpec((_H, 1), lambda i: (0, 0)),
            pl.BlockSpec((1, 1), lambda i: (0, 0)),
        ],
        out_specs=pl.BlockSpec((_BB, _N), lambda i: (jnp.maximum(i - 1, 0), 0)),
        out_shape=jax.ShapeDtypeStruct((nb, _N), jnp.float32),
        scratch_shapes=[pltpu.VMEM((rows, _H), jnp.float32)],
    )(x_flat, W1, b1.reshape(1, _H), W2, b2.reshape(1, 1))


def _topk_body(s_hbm, out_hbm, s_v, out_v):
    wid = lax.axis_index("s") * 2 + lax.axis_index("c")
    rows = s_v.shape[0]
    base = wid * rows
    pltpu.sync_copy(s_hbm.at[pl.ds(base, rows)], s_v)
    zero = jnp.zeros((_L,), jnp.float32)
    for r in range(rows):
        ka = ia = None
        for c in range(_N // _L):
            vals = s_v[r, pl.ds(c * _L, _L)]
            kb = jnp.abs(vals)
            ib = lax.iota(jnp.int32, _L) + c * _L
            if c == 0:
                ka, ia = plsc.sort_key_val(kb, ib)
            else:
                kb, ib = plsc.sort_key_val(kb, ib, descending=True)
                take_a = ka >= kb
                km = jnp.where(take_a, ka, kb)
                im = jnp.where(take_a, ia, ib)
                ka, ia = plsc.sort_key_val(km, im)
        denom = jnp.sum(ka) + 1e-8
        row_splat = jnp.full((_L,), r, jnp.int32)
        sel = plsc.load_gather(s_v, [row_splat, ia])
        for c in range(_N // _L):
            out_v[r, pl.ds(c * _L, _L)] = zero
        plsc.store_scatter(out_v, [row_splat, ia], sel / denom)
    pltpu.sync_copy(out_v, out_hbm.at[pl.ds(base, rows)])


def _topk_sc(s2d):
    nb = s2d.shape[0]
    rows = nb // _NW
    f = pl.kernel(
        _topk_body,
        out_type=jax.ShapeDtypeStruct((nb, _N), jnp.float32),
        mesh=plsc.VectorSubcoreMesh(core_axis_name="c", subcore_axis_name="s"),
        compiler_params=pltpu.CompilerParams(needs_layout_passes=False),
        scratch_types=[
            pltpu.VMEM((rows, _N), jnp.float32),
            pltpu.VMEM((rows, _N), jnp.float32),
        ],
    )
    return f(s2d)


def kernel(signal_features, W1, b1, W2, b2):
    x_flat = signal_features.reshape(_B * _N, _D)
    s = _mlp_scores(x_flat, W1, b1, W2, b2, _B)
    return _topk_sc(s)
